# baseline (speedup 1.0000x reference)
import functools

import jax
import jax.numpy as jnp
from jax import lax
from jax.experimental import pallas as pl
from jax.experimental.pallas import tpu as pltpu
from jax.experimental.pallas import tpu_sc as plsc

VAR_WEIGHT = 1.0
DIST_WEIGHT = 1.0
MARGIN_VAR = 0.1
MARGIN_DIST = 1.5
MAX_LABEL = 16

BATCH = 8
GROUP = 4
NV = 1024

ROW_W = 256


@functools.cache
def _build_sc_kernel():
  mesh = plsc.VectorSubcoreMesh(
      core_axis_name="c", subcore_axis_name="s", num_cores=2, num_subcores=16)

  @functools.partial(
      pl.kernel,
      out_type=jax.ShapeDtypeStruct((BATCH, ROW_W), jnp.float32),
      mesh=mesh,
      compiler_params=pltpu.CompilerParams(needs_layout_passes=False),
      scratch_types=[
          pltpu.VMEM((32, 512), jnp.float32),
          pltpu.VMEM((32, 512), jnp.int32),
          pltpu.VMEM((32, 512), jnp.float32),
          pltpu.VMEM((32, 512), jnp.int32),
          pltpu.VMEM((64, 256), jnp.float32),
          pltpu.VMEM((64, 256), jnp.int32),
          pltpu.VMEM((544,), jnp.float32),
          pltpu.VMEM((544,), jnp.float32),
          pltpu.VMEM((544,), jnp.float32),
          pltpu.VMEM((544,), jnp.float32),
          pltpu.VMEM((544,), jnp.float32),
          pltpu.VMEM((544,), jnp.float32),
          pltpu.VMEM((544,), jnp.float32),
          pltpu.VMEM((544,), jnp.float32),
          pltpu.VMEM((544,), jnp.float32),
          pltpu.VMEM((544,), jnp.float32),
          pltpu.VMEM((544,), jnp.float32),
          pltpu.VMEM((ROW_W,), jnp.float32),
          pltpu.VMEM((ROW_W,), jnp.float32),
          pltpu.VMEM((GROUP, ROW_W), jnp.float32),
          pltpu.VMEM_SHARED((16, ROW_W), jnp.float32),
          pltpu.SemaphoreType.DMA,
          pltpu.SemaphoreType.DMA,
          pltpu.SemaphoreType.DMA,
          pltpu.SemaphoreType.DMA,
          pltpu.SemaphoreType.DMA,
          pltpu.SemaphoreType.DMA,
      ],
  )
  def sc_kernel(f0, g0, f1, g1, out, fb0a, gb0a, fb0b, gb0b, fb1, gb1,
                cnt0e, cnt0o, cnt1e, cnt1o, fsume, fsumo, dsum0e, dsum0o,
                dsum1e, dsum1o, meantab, pub, rowbuf, grp,
                shared, semf0a, semg0a, semf0b, semg0b, semf1, semg1):
    fbufs = (fb0a, fb0b)
    gbufs = (gb0a, gb0b)
    semsf = (semf0a, semf0b)
    semsg = (semg0a, semg0b)
    c = lax.axis_index("c")
    s = lax.axis_index("s")
    lb = s // GROUP
    m = s % GROUP
    b = c * 4 + lb
    lane = lax.iota(jnp.int32, 16)
    lane33 = lane * 33
    zeros16 = jnp.zeros((16,), jnp.float32)
    ones16 = jnp.ones((16,), jnp.float32)

    for tab in (cnt0e, cnt0o, cnt1e, cnt1o, fsume, fsumo,
                dsum0e, dsum0o, dsum1e, dsum1o):
      for r in range(34):
        tab[pl.ds(r * 16, 16)] = zeros16
    for k in range(ROW_W // 16):
      pub[pl.ds(k * 16, 16)] = zeros16
      rowbuf[pl.ds(k * 16, 16)] = zeros16

    def start0(k):
      roff = m * 128 + k * 32
      bi = k % 2
      return (pltpu.async_copy(f0.at[b, 0, pl.ds(roff, 32), :],
                               fbufs[bi], semsf[bi]),
              pltpu.async_copy(g0.at[b, 0, pl.ds(roff, 32), :],
                               gbufs[bi], semsg[bi]))

    def stream(body0, body1, d0, stream1):
      if stream1:
        d1f = pltpu.async_copy(f1.at[b, 0, pl.ds(m * 64, 64), :], fb1, semf1)
        d1g = pltpu.async_copy(g1.at[b, 0, pl.ds(m * 64, 64), :], gb1, semg1)
      d = d0
      for k in range(4):
        nd = start0(k + 1) if k < 3 else None
        d[0].wait()
        d[1].wait()
        body0(fbufs[k % 2], gbufs[k % 2])
        d = nd
      if stream1:
        d1f.wait()
        d1g.wait()
      body1(fb1, gb1)

    def body_a(cnte, cnto, shift):
      mask = (1 << shift) - 1
      def run(fbuf, gbuf):
        @plsc.parallel_loop(0, NV, step=2, unroll=4)
        def _(i):
          for (k, cnt, fs) in ((i, cnte, fsume), (i + 1, cnto, fsumo)):
            ri = k >> shift
            ci = (k & mask) * 16
            gv = gbuf[ri, pl.ds(ci, 16)]
            fv = fbuf[ri, pl.ds(ci, 16)]
            idx = lane33 + gv
            plsc.addupdate_scatter(cnt, [idx], ones16)
            plsc.addupdate_scatter(fs, [idx], fv)
      return run

    stream(body_a(cnt0e, cnt0o, 5), body_a(cnt1e, cnt1o, 4), start0(0), True)

    def reduce_tab(tabe, tabo, h):
      acc = tabe[pl.ds(h * 16, 16)] + tabo[pl.ds(h * 16, 16)]
      for r in range(1, 16):
        acc = acc + tabe[pl.ds(r * 33 + h * 16, 16)]
        acc = acc + tabo[pl.ds(r * 33 + h * 16, 16)]
      return acc

    db0 = start0(0)
    for h in range(2):
      pub[pl.ds(0 + h * 16, 16)] = reduce_tab(cnt0e, cnt0o, h)
      pub[pl.ds(32 + h * 16, 16)] = reduce_tab(cnt1e, cnt1o, h)
      pub[pl.ds(64 + h * 16, 16)] = reduce_tab(fsume, fsumo, h)
    pltpu.sync_copy(pub, shared.at[s])
    plsc.subcore_barrier()

    pltpu.sync_copy(shared.at[pl.ds(lb * GROUP, GROUP)], grp)
    for h in range(2):
      c0c = grp[0, pl.ds(0 + h * 16, 16)]
      c1c = grp[0, pl.ds(32 + h * 16, 16)]
      sfc = grp[0, pl.ds(64 + h * 16, 16)]
      for mm in range(1, GROUP):
        c0c = c0c + grp[mm, pl.ds(0 + h * 16, 16)]
        c1c = c1c + grp[mm, pl.ds(32 + h * 16, 16)]
        sfc = sfc + grp[mm, pl.ds(64 + h * 16, 16)]
      tot = c0c + c1c
      mh = sfc / jnp.maximum(tot, 1.0)
      for l in range(16):
        meantab[pl.ds(l * 33 + h * 16, 16)] = mh
      rowbuf[pl.ds(0 + h * 16, 16)] = c0c
      rowbuf[pl.ds(32 + h * 16, 16)] = c1c
      rowbuf[pl.ds(64 + h * 16, 16)] = sfc

    def body_b(dte, dto, shift):
      mask = (1 << shift) - 1
      def run(fbuf, gbuf):
        @plsc.parallel_loop(0, NV, step=2, unroll=4)
        def _(i):
          for (k, dt) in ((i, dte), (i + 1, dto)):
            ri = k >> shift
            ci = (k & mask) * 16
            gv = gbuf[ri, pl.ds(ci, 16)]
            fv = fbuf[ri, pl.ds(ci, 16)]
            idx = lane33 + gv
            mv = plsc.load_gather(meantab, [idx])
            d = jnp.maximum(jnp.abs(fv - mv) - MARGIN_VAR, 0.0)
            plsc.addupdate_scatter(dt, [idx], d * d)
      return run

    stream(body_b(dsum0e, dsum0o, 5), body_b(dsum1e, dsum1o, 4), db0, False)

    for h in range(2):
      pub[pl.ds(96 + h * 16, 16)] = reduce_tab(dsum0e, dsum0o, h)
      pub[pl.ds(128 + h * 16, 16)] = reduce_tab(dsum1e, dsum1o, h)
    pltpu.sync_copy(pub, shared.at[s])
    plsc.subcore_barrier()

    @pl.when(m == 0)
    def _():
      pltpu.sync_copy(shared.at[pl.ds(lb * GROUP, GROUP)], grp)
      for h in range(2):
        d0c = grp[0, pl.ds(96 + h * 16, 16)]
        d1c = grp[0, pl.ds(128 + h * 16, 16)]
        for mm in range(1, GROUP):
          d0c = d0c + grp[mm, pl.ds(96 + h * 16, 16)]
          d1c = d1c + grp[mm, pl.ds(128 + h * 16, 16)]
        rowbuf[pl.ds(96 + h * 16, 16)] = d0c
        rowbuf[pl.ds(128 + h * 16, 16)] = d1c
      pltpu.sync_copy(rowbuf, out.at[b])

  return sc_kernel


def _epilogue_body(x_ref, o_ref):
  x = x_ref[...]
  c0 = x[:, 0:32]
  c1 = x[:, 32:64]
  sf = x[:, 64:96]
  d0 = x[:, 96:128]
  d1 = x[:, 128:160]
  lanei = lax.broadcasted_iota(jnp.int32, (BATCH, 32), 1)
  tot = c0 + c1
  mean = sf / jnp.maximum(tot, 1.0)
  pres0 = jnp.sum(c0, axis=0, keepdims=True) > 0.0
  big_c = jnp.max(jnp.where(pres0, lanei[0:1, :], 0))
  valid = (tot > 0.0) & (lanei >= 1) & (lanei <= MAX_LABEL) & (lanei <= big_c)
  validf = jnp.where(valid, 1.0, 0.0)
  pull_val = (jnp.where(c0 > 0.0, d0 / jnp.maximum(c0, 1.0), 0.0)
              + jnp.where(c1 > 0.0, d1 / jnp.maximum(c1, 1.0), 0.0))
  pull_sum = jnp.sum(pull_val * validf)
  pull_cnt = jnp.sum(validf)
  push_sum = jnp.float32(0.0)
  push_cnt = jnp.float32(0.0)
  for i in range(1, MAX_LABEL + 1):
    mi = mean[:, i:i + 1]
    vi = validf[:, i:i + 1]
    pv = vi * validf * jnp.where(lanei != i, 1.0, 0.0)
    il = jnp.maximum(2.0 * MARGIN_DIST - jnp.abs(mean - mi), 0.0)
    push_sum = push_sum + jnp.sum(pv * il * il)
    push_cnt = push_cnt + jnp.sum(pv)
  pull = jnp.where(pull_cnt > 0.0,
                   pull_sum / jnp.maximum(pull_cnt, 1.0) * VAR_WEIGHT, 0.0)
  push = jnp.where(push_cnt > 0.0,
                   push_sum / jnp.maximum(push_cnt, 1.0) * DIST_WEIGHT, 0.0)
  o_ref[...] = jnp.full((1, 1), pull + push, jnp.float32)


@jax.jit
def _run(f0, g0, f1, g1):
  rows = _build_sc_kernel()(f0, g0, f1, g1)
  loss = pl.pallas_call(
      _epilogue_body,
      out_shape=jax.ShapeDtypeStruct((1, 1), jnp.float32),
  )(rows)
  return jnp.reshape(loss, ())


def kernel(featmap_s0, featmap_s1, gt_s0, gt_s1):
  g0 = gt_s0.astype(jnp.int32)
  g1 = gt_s1.astype(jnp.int32)
  return _run(featmap_s0, g0, featmap_s1, g1)


# speedup vs baseline: 1.0240x; 1.0240x over previous
import functools

import jax
import jax.numpy as jnp
from jax import lax
from jax.experimental import pallas as pl
from jax.experimental.pallas import tpu as pltpu
from jax.experimental.pallas import tpu_sc as plsc

VAR_WEIGHT = 1.0
DIST_WEIGHT = 1.0
MARGIN_VAR = 0.1
MARGIN_DIST = 1.5
MAX_LABEL = 16

BATCH = 8
GROUP = 4
NV = 1024

ROW_W = 256


@functools.cache
def _build_sc_kernel():
  mesh = plsc.VectorSubcoreMesh(
      core_axis_name="c", subcore_axis_name="s", num_cores=2, num_subcores=16)

  @functools.partial(
      pl.kernel,
      out_type=jax.ShapeDtypeStruct((BATCH, ROW_W), jnp.float32),
      mesh=mesh,
      compiler_params=pltpu.CompilerParams(needs_layout_passes=False),
      scratch_types=[
          pltpu.VMEM((32, 512), jnp.float32),
          pltpu.VMEM((32, 512), jnp.int32),
          pltpu.VMEM((32, 512), jnp.float32),
          pltpu.VMEM((32, 512), jnp.int32),
          pltpu.VMEM((64, 256), jnp.float32),
          pltpu.VMEM((64, 256), jnp.int32),
          pltpu.VMEM((544,), jnp.float32),
          pltpu.VMEM((544,), jnp.float32),
          pltpu.VMEM((544,), jnp.float32),
          pltpu.VMEM((544,), jnp.float32),
          pltpu.VMEM((544,), jnp.float32),
          pltpu.VMEM((544,), jnp.float32),
          pltpu.VMEM((ROW_W,), jnp.float32),
          pltpu.VMEM((ROW_W,), jnp.float32),
          pltpu.VMEM((GROUP, ROW_W), jnp.float32),
          pltpu.VMEM_SHARED((16, ROW_W), jnp.float32),
          pltpu.SemaphoreType.DMA,
          pltpu.SemaphoreType.DMA,
          pltpu.SemaphoreType.DMA,
          pltpu.SemaphoreType.DMA,
          pltpu.SemaphoreType.DMA,
          pltpu.SemaphoreType.DMA,
      ],
  )
  def sc_kernel(f0, g0, f1, g1, out, fb0a, gb0a, fb0b, gb0b, fb1, gb1,
                cnt0, cnt1, fsum, dsum0, dsum1, meantab, pub, rowbuf, grp,
                shared, semf0a, semg0a, semf0b, semg0b, semf1, semg1):
    fbufs = (fb0a, fb0b)
    gbufs = (gb0a, gb0b)
    semsf = (semf0a, semf0b)
    semsg = (semg0a, semg0b)
    c = lax.axis_index("c")
    s = lax.axis_index("s")
    lb = s // GROUP
    m = s % GROUP
    b = c * 4 + lb
    lane = lax.iota(jnp.int32, 16)
    lane33 = lane * 33
    zeros16 = jnp.zeros((16,), jnp.float32)
    ones16 = jnp.ones((16,), jnp.float32)

    for tab in (cnt0, cnt1, fsum, dsum0, dsum1):
      for r in range(34):
        tab[pl.ds(r * 16, 16)] = zeros16
    for k in range(ROW_W // 16):
      pub[pl.ds(k * 16, 16)] = zeros16
      rowbuf[pl.ds(k * 16, 16)] = zeros16

    def start0(k):
      roff = m * 128 + k * 32
      bi = k % 2
      return (pltpu.async_copy(f0.at[b, 0, pl.ds(roff, 32), :],
                               fbufs[bi], semsf[bi]),
              pltpu.async_copy(g0.at[b, 0, pl.ds(roff, 32), :],
                               gbufs[bi], semsg[bi]))

    def stream(body0, body1, d0, stream1):
      if stream1:
        d1f = pltpu.async_copy(f1.at[b, 0, pl.ds(m * 64, 64), :], fb1, semf1)
        d1g = pltpu.async_copy(g1.at[b, 0, pl.ds(m * 64, 64), :], gb1, semg1)
      d = d0
      for k in range(4):
        nd = start0(k + 1) if k < 3 else None
        d[0].wait()
        d[1].wait()
        body0(fbufs[k % 2], gbufs[k % 2])
        d = nd
      if stream1:
        d1f.wait()
        d1g.wait()
      body1(fb1, gb1)

    def body_a(cnt, shift):
      mask = (1 << shift) - 1
      def run(fbuf, gbuf):
        @plsc.parallel_loop(0, NV, unroll=8)
        def _(i):
          ri = i >> shift
          ci = (i & mask) * 16
          gv = gbuf[ri, pl.ds(ci, 16)]
          fv = fbuf[ri, pl.ds(ci, 16)]
          idx = lane33 + gv
          plsc.addupdate_scatter(cnt, [idx], ones16)
          plsc.addupdate_scatter(fsum, [idx], fv)
      return run

    stream(body_a(cnt0, 5), body_a(cnt1, 4), start0(0), True)

    def reduce_tab(tab, h):
      acc = tab[pl.ds(h * 16, 16)]
      for r in range(1, 16):
        acc = acc + tab[pl.ds(r * 33 + h * 16, 16)]
      return acc

    db0 = start0(0)
    for h in range(2):
      pub[pl.ds(0 + h * 16, 16)] = reduce_tab(cnt0, h)
      pub[pl.ds(32 + h * 16, 16)] = reduce_tab(cnt1, h)
      pub[pl.ds(64 + h * 16, 16)] = reduce_tab(fsum, h)
    pltpu.sync_copy(pub, shared.at[s])
    plsc.subcore_barrier()

    pltpu.sync_copy(shared.at[pl.ds(lb * GROUP, GROUP)], grp)
    for h in range(2):
      c0c = grp[0, pl.ds(0 + h * 16, 16)]
      c1c = grp[0, pl.ds(32 + h * 16, 16)]
      sfc = grp[0, pl.ds(64 + h * 16, 16)]
      for mm in range(1, GROUP):
        c0c = c0c + grp[mm, pl.ds(0 + h * 16, 16)]
        c1c = c1c + grp[mm, pl.ds(32 + h * 16, 16)]
        sfc = sfc + grp[mm, pl.ds(64 + h * 16, 16)]
      tot = c0c + c1c
      mh = sfc / jnp.maximum(tot, 1.0)
      for l in range(16):
        meantab[pl.ds(l * 33 + h * 16, 16)] = mh
      rowbuf[pl.ds(0 + h * 16, 16)] = c0c
      rowbuf[pl.ds(32 + h * 16, 16)] = c1c
      rowbuf[pl.ds(64 + h * 16, 16)] = sfc

    def body_b(dtab, shift):
      mask = (1 << shift) - 1
      def run(fbuf, gbuf):
        @plsc.parallel_loop(0, NV, unroll=8)
        def _(i):
          ri = i >> shift
          ci = (i & mask) * 16
          gv = gbuf[ri, pl.ds(ci, 16)]
          fv = fbuf[ri, pl.ds(ci, 16)]
          idx = lane33 + gv
          mv = plsc.load_gather(meantab, [idx])
          d = jnp.maximum(jnp.abs(fv - mv) - MARGIN_VAR, 0.0)
          plsc.addupdate_scatter(dtab, [idx], d * d)
      return run

    stream(body_b(dsum0, 5), body_b(dsum1, 4), db0, False)

    for h in range(2):
      pub[pl.ds(96 + h * 16, 16)] = reduce_tab(dsum0, h)
      pub[pl.ds(128 + h * 16, 16)] = reduce_tab(dsum1, h)
    pltpu.sync_copy(pub, shared.at[s])
    plsc.subcore_barrier()

    @pl.when(m == 0)
    def _():
      pltpu.sync_copy(shared.at[pl.ds(lb * GROUP, GROUP)], grp)
      for h in range(2):
        d0c = grp[0, pl.ds(96 + h * 16, 16)]
        d1c = grp[0, pl.ds(128 + h * 16, 16)]
        for mm in range(1, GROUP):
          d0c = d0c + grp[mm, pl.ds(96 + h * 16, 16)]
          d1c = d1c + grp[mm, pl.ds(128 + h * 16, 16)]
        rowbuf[pl.ds(96 + h * 16, 16)] = d0c
        rowbuf[pl.ds(128 + h * 16, 16)] = d1c
      pltpu.sync_copy(rowbuf, out.at[b])

  return sc_kernel


def _epilogue_body(x_ref, o_ref):
  x = x_ref[...]
  c0 = x[:, 0:32]
  c1 = x[:, 32:64]
  sf = x[:, 64:96]
  d0 = x[:, 96:128]
  d1 = x[:, 128:160]
  lanei = lax.broadcasted_iota(jnp.int32, (BATCH, 32), 1)
  tot = c0 + c1
  mean = sf / jnp.maximum(tot, 1.0)
  pres0 = jnp.sum(c0, axis=0, keepdims=True) > 0.0
  big_c = jnp.max(jnp.where(pres0, lanei[0:1, :], 0))
  valid = (tot > 0.0) & (lanei >= 1) & (lanei <= MAX_LABEL) & (lanei <= big_c)
  validf = jnp.where(valid, 1.0, 0.0)
  pull_val = (jnp.where(c0 > 0.0, d0 / jnp.maximum(c0, 1.0), 0.0)
              + jnp.where(c1 > 0.0, d1 / jnp.maximum(c1, 1.0), 0.0))
  pull_sum = jnp.sum(pull_val * validf)
  pull_cnt = jnp.sum(validf)
  push_sum = jnp.float32(0.0)
  push_cnt = jnp.float32(0.0)
  for i in range(1, MAX_LABEL + 1):
    mi = mean[:, i:i + 1]
    vi = validf[:, i:i + 1]
    pv = vi * validf * jnp.where(lanei != i, 1.0, 0.0)
    il = jnp.maximum(2.0 * MARGIN_DIST - jnp.abs(mean - mi), 0.0)
    push_sum = push_sum + jnp.sum(pv * il * il)
    push_cnt = push_cnt + jnp.sum(pv)
  pull = jnp.where(pull_cnt > 0.0,
                   pull_sum / jnp.maximum(pull_cnt, 1.0) * VAR_WEIGHT, 0.0)
  push = jnp.where(push_cnt > 0.0,
                   push_sum / jnp.maximum(push_cnt, 1.0) * DIST_WEIGHT, 0.0)
  o_ref[...] = jnp.full((1, 1), pull + push, jnp.float32)


@jax.jit
def _run(f0, g0, f1, g1):
  rows = _build_sc_kernel()(f0, g0, f1, g1)
  loss = pl.pallas_call(
      _epilogue_body,
      out_shape=jax.ShapeDtypeStruct((1, 1), jnp.float32),
  )(rows)
  return jnp.reshape(loss, ())


def kernel(featmap_s0, featmap_s1, gt_s0, gt_s1):
  g0 = gt_s0.astype(jnp.int32)
  g1 = gt_s1.astype(jnp.int32)
  return _run(featmap_s0, g0, featmap_s1, g1)


# speedup vs baseline: 1.0343x; 1.0101x over previous
import functools

import jax
import jax.numpy as jnp
from jax import lax
from jax.experimental import pallas as pl
from jax.experimental.pallas import tpu as pltpu
from jax.experimental.pallas import tpu_sc as plsc

VAR_WEIGHT = 1.0
DIST_WEIGHT = 1.0
MARGIN_VAR = 0.1
MARGIN_DIST = 1.5
MAX_LABEL = 16

BATCH = 8
GROUP = 4
NV = 1024

ROW_W = 256


@functools.cache
def _build_sc_kernel():
  mesh = plsc.VectorSubcoreMesh(
      core_axis_name="c", subcore_axis_name="s", num_cores=2, num_subcores=16)

  @functools.partial(
      pl.kernel,
      out_type=jax.ShapeDtypeStruct((BATCH, ROW_W), jnp.float32),
      mesh=mesh,
      compiler_params=pltpu.CompilerParams(needs_layout_passes=False),
      scratch_types=[
          pltpu.VMEM((16, 512), jnp.float32),
          pltpu.VMEM((16, 512), jnp.int32),
          pltpu.VMEM((16, 512), jnp.float32),
          pltpu.VMEM((16, 512), jnp.int32),
          pltpu.VMEM((64, 256), jnp.float32),
          pltpu.VMEM((64, 256), jnp.int32),
          pltpu.VMEM((544,), jnp.float32),
          pltpu.VMEM((544,), jnp.float32),
          pltpu.VMEM((544,), jnp.float32),
          pltpu.VMEM((544,), jnp.float32),
          pltpu.VMEM((544,), jnp.float32),
          pltpu.VMEM((544,), jnp.float32),
          pltpu.VMEM((ROW_W,), jnp.float32),
          pltpu.VMEM((ROW_W,), jnp.float32),
          pltpu.VMEM((GROUP, ROW_W), jnp.float32),
          pltpu.VMEM_SHARED((16, ROW_W), jnp.float32),
          pltpu.SemaphoreType.DMA,
          pltpu.SemaphoreType.DMA,
          pltpu.SemaphoreType.DMA,
          pltpu.SemaphoreType.DMA,
          pltpu.SemaphoreType.DMA,
          pltpu.SemaphoreType.DMA,
      ],
  )
  def sc_kernel(f0, g0, f1, g1, out, fb0a, gb0a, fb0b, gb0b, fb1, gb1,
                cnt0, cnt1, fsum, dsum0, dsum1, meantab, pub, rowbuf, grp,
                shared, semf0a, semg0a, semf0b, semg0b, semf1, semg1):
    fbufs = (fb0a, fb0b)
    gbufs = (gb0a, gb0b)
    semsf = (semf0a, semf0b)
    semsg = (semg0a, semg0b)
    c = lax.axis_index("c")
    s = lax.axis_index("s")
    lb = s // GROUP
    m = s % GROUP
    b = c * 4 + lb
    lane = lax.iota(jnp.int32, 16)
    lane33 = lane * 33
    zeros16 = jnp.zeros((16,), jnp.float32)
    ones16 = jnp.ones((16,), jnp.float32)

    for tab in (cnt0, cnt1, fsum, dsum0, dsum1):
      for r in range(34):
        tab[pl.ds(r * 16, 16)] = zeros16
    for k in range(ROW_W // 16):
      pub[pl.ds(k * 16, 16)] = zeros16
      rowbuf[pl.ds(k * 16, 16)] = zeros16

    def start0(k):
      roff = m * 128 + k * 16
      bi = k % 2
      return (pltpu.async_copy(f0.at[b, 0, pl.ds(roff, 16), :],
                               fbufs[bi], semsf[bi]),
              pltpu.async_copy(g0.at[b, 0, pl.ds(roff, 16), :],
                               gbufs[bi], semsg[bi]))

    def stream(body0, body1, d0, stream1):
      if stream1:
        d1f = pltpu.async_copy(f1.at[b, 0, pl.ds(m * 64, 64), :], fb1, semf1)
        d1g = pltpu.async_copy(g1.at[b, 0, pl.ds(m * 64, 64), :], gb1, semg1)
      d = d0
      for k in range(8):
        nd = start0(k + 1) if k < 7 else None
        d[0].wait()
        d[1].wait()
        body0(fbufs[k % 2], gbufs[k % 2])
        d = nd
      if stream1:
        d1f.wait()
        d1g.wait()
      body1(fb1, gb1)

    def body_a(cnt, shift, nv=NV):
      mask = (1 << shift) - 1
      def run(fbuf, gbuf):
        @plsc.parallel_loop(0, nv, unroll=8)
        def _(i):
          ri = i >> shift
          ci = (i & mask) * 16
          gv = gbuf[ri, pl.ds(ci, 16)]
          fv = fbuf[ri, pl.ds(ci, 16)]
          idx = lane33 + gv
          plsc.addupdate_scatter(cnt, [idx], ones16)
          plsc.addupdate_scatter(fsum, [idx], fv)
      return run

    stream(body_a(cnt0, 5, 512), body_a(cnt1, 4), start0(0), True)

    def reduce_tab(tab, h):
      acc = tab[pl.ds(h * 16, 16)]
      for r in range(1, 16):
        acc = acc + tab[pl.ds(r * 33 + h * 16, 16)]
      return acc

    db0 = start0(0)
    for h in range(2):
      pub[pl.ds(0 + h * 16, 16)] = reduce_tab(cnt0, h)
      pub[pl.ds(32 + h * 16, 16)] = reduce_tab(cnt1, h)
      pub[pl.ds(64 + h * 16, 16)] = reduce_tab(fsum, h)
    pltpu.sync_copy(pub, shared.at[s])
    plsc.subcore_barrier()

    pltpu.sync_copy(shared.at[pl.ds(lb * GROUP, GROUP)], grp)
    for h in range(2):
      c0c = grp[0, pl.ds(0 + h * 16, 16)]
      c1c = grp[0, pl.ds(32 + h * 16, 16)]
      sfc = grp[0, pl.ds(64 + h * 16, 16)]
      for mm in range(1, GROUP):
        c0c = c0c + grp[mm, pl.ds(0 + h * 16, 16)]
        c1c = c1c + grp[mm, pl.ds(32 + h * 16, 16)]
        sfc = sfc + grp[mm, pl.ds(64 + h * 16, 16)]
      tot = c0c + c1c
      mh = sfc / jnp.maximum(tot, 1.0)
      for l in range(16):
        meantab[pl.ds(l * 33 + h * 16, 16)] = mh
      rowbuf[pl.ds(0 + h * 16, 16)] = c0c
      rowbuf[pl.ds(32 + h * 16, 16)] = c1c
      rowbuf[pl.ds(64 + h * 16, 16)] = sfc

    def body_b(dtab, shift, nv=NV):
      mask = (1 << shift) - 1
      def run(fbuf, gbuf):
        @plsc.parallel_loop(0, nv, unroll=8)
        def _(i):
          ri = i >> shift
          ci = (i & mask) * 16
          gv = gbuf[ri, pl.ds(ci, 16)]
          fv = fbuf[ri, pl.ds(ci, 16)]
          idx = lane33 + gv
          mv = plsc.load_gather(meantab, [idx])
          d = jnp.maximum(jnp.abs(fv - mv) - MARGIN_VAR, 0.0)
          plsc.addupdate_scatter(dtab, [idx], d * d)
      return run

    stream(body_b(dsum0, 5, 512), body_b(dsum1, 4), db0, False)

    for h in range(2):
      pub[pl.ds(96 + h * 16, 16)] = reduce_tab(dsum0, h)
      pub[pl.ds(128 + h * 16, 16)] = reduce_tab(dsum1, h)
    pltpu.sync_copy(pub, shared.at[s])
    plsc.subcore_barrier()

    @pl.when(m == 0)
    def _():
      pltpu.sync_copy(shared.at[pl.ds(lb * GROUP, GROUP)], grp)
      for h in range(2):
        d0c = grp[0, pl.ds(96 + h * 16, 16)]
        d1c = grp[0, pl.ds(128 + h * 16, 16)]
        for mm in range(1, GROUP):
          d0c = d0c + grp[mm, pl.ds(96 + h * 16, 16)]
          d1c = d1c + grp[mm, pl.ds(128 + h * 16, 16)]
        rowbuf[pl.ds(96 + h * 16, 16)] = d0c
        rowbuf[pl.ds(128 + h * 16, 16)] = d1c
      pltpu.sync_copy(rowbuf, out.at[b])

  return sc_kernel


def _epilogue_body(x_ref, o_ref):
  x = x_ref[...]
  c0 = x[:, 0:32]
  c1 = x[:, 32:64]
  sf = x[:, 64:96]
  d0 = x[:, 96:128]
  d1 = x[:, 128:160]
  lanei = lax.broadcasted_iota(jnp.int32, (BATCH, 32), 1)
  tot = c0 + c1
  mean = sf / jnp.maximum(tot, 1.0)
  pres0 = jnp.sum(c0, axis=0, keepdims=True) > 0.0
  big_c = jnp.max(jnp.where(pres0, lanei[0:1, :], 0))
  valid = (tot > 0.0) & (lanei >= 1) & (lanei <= MAX_LABEL) & (lanei <= big_c)
  validf = jnp.where(valid, 1.0, 0.0)
  pull_val = (jnp.where(c0 > 0.0, d0 / jnp.maximum(c0, 1.0), 0.0)
              + jnp.where(c1 > 0.0, d1 / jnp.maximum(c1, 1.0), 0.0))
  pull_sum = jnp.sum(pull_val * validf)
  pull_cnt = jnp.sum(validf)
  push_sum = jnp.float32(0.0)
  push_cnt = jnp.float32(0.0)
  for i in range(1, MAX_LABEL + 1):
    mi = mean[:, i:i + 1]
    vi = validf[:, i:i + 1]
    pv = vi * validf * jnp.where(lanei != i, 1.0, 0.0)
    il = jnp.maximum(2.0 * MARGIN_DIST - jnp.abs(mean - mi), 0.0)
    push_sum = push_sum + jnp.sum(pv * il * il)
    push_cnt = push_cnt + jnp.sum(pv)
  pull = jnp.where(pull_cnt > 0.0,
                   pull_sum / jnp.maximum(pull_cnt, 1.0) * VAR_WEIGHT, 0.0)
  push = jnp.where(push_cnt > 0.0,
                   push_sum / jnp.maximum(push_cnt, 1.0) * DIST_WEIGHT, 0.0)
  o_ref[...] = jnp.full((1, 1), pull + push, jnp.float32)


@jax.jit
def _run(f0, g0, f1, g1):
  rows = _build_sc_kernel()(f0, g0, f1, g1)
  loss = pl.pallas_call(
      _epilogue_body,
      out_shape=jax.ShapeDtypeStruct((1, 1), jnp.float32),
  )(rows)
  return jnp.reshape(loss, ())


def kernel(featmap_s0, featmap_s1, gt_s0, gt_s1):
  g0 = gt_s0.astype(jnp.int32)
  g1 = gt_s1.astype(jnp.int32)
  return _run(featmap_s0, g0, featmap_s1, g1)


# speedup vs baseline: 1.0344x; 1.0001x over previous
import functools

import jax
import jax.numpy as jnp
from jax import lax
from jax.experimental import pallas as pl
from jax.experimental.pallas import tpu as pltpu
from jax.experimental.pallas import tpu_sc as plsc

VAR_WEIGHT = 1.0
DIST_WEIGHT = 1.0
MARGIN_VAR = 0.1
MARGIN_DIST = 1.5
MAX_LABEL = 16

BATCH = 8
GROUP = 4
NV = 1024

ROW_W = 256


@functools.cache
def _build_sc_kernel():
  mesh = plsc.VectorSubcoreMesh(
      core_axis_name="c", subcore_axis_name="s", num_cores=2, num_subcores=16)

  @functools.partial(
      pl.kernel,
      out_type=jax.ShapeDtypeStruct((BATCH, ROW_W), jnp.float32),
      mesh=mesh,
      compiler_params=pltpu.CompilerParams(needs_layout_passes=False),
      scratch_types=[
          pltpu.VMEM((16, 512), jnp.float32),
          pltpu.VMEM((16, 512), jnp.int32),
          pltpu.VMEM((16, 512), jnp.float32),
          pltpu.VMEM((16, 512), jnp.int32),
          pltpu.VMEM((16, 512), jnp.float32),
          pltpu.VMEM((16, 512), jnp.int32),
          pltpu.VMEM((64, 256), jnp.float32),
          pltpu.VMEM((64, 256), jnp.int32),
          pltpu.VMEM((544,), jnp.float32),
          pltpu.VMEM((544,), jnp.float32),
          pltpu.VMEM((544,), jnp.float32),
          pltpu.VMEM((544,), jnp.float32),
          pltpu.VMEM((544,), jnp.float32),
          pltpu.VMEM((544,), jnp.float32),
          pltpu.VMEM((ROW_W,), jnp.float32),
          pltpu.VMEM((ROW_W,), jnp.float32),
          pltpu.VMEM((GROUP, ROW_W), jnp.float32),
          pltpu.VMEM_SHARED((16, ROW_W), jnp.float32),
          pltpu.SemaphoreType.DMA,
          pltpu.SemaphoreType.DMA,
          pltpu.SemaphoreType.DMA,
          pltpu.SemaphoreType.DMA,
          pltpu.SemaphoreType.DMA,
          pltpu.SemaphoreType.DMA,
          pltpu.SemaphoreType.DMA,
          pltpu.SemaphoreType.DMA,
      ],
  )
  def sc_kernel(f0, g0, f1, g1, out, fb0a, gb0a, fb0b, gb0b, fb0c, gb0c,
                fb1, gb1, cnt0, cnt1, fsum, dsum0, dsum1, meantab, pub,
                rowbuf, grp, shared, semf0a, semg0a, semf0b, semg0b,
                semf0c, semg0c, semf1, semg1):
    fbufs = (fb0a, fb0b, fb0c)
    gbufs = (gb0a, gb0b, gb0c)
    semsf = (semf0a, semf0b, semf0c)
    semsg = (semg0a, semg0b, semg0c)
    c = lax.axis_index("c")
    s = lax.axis_index("s")
    lb = s // GROUP
    m = s % GROUP
    b = c * 4 + lb
    lane = lax.iota(jnp.int32, 16)
    lane33 = lane * 33
    zeros16 = jnp.zeros((16,), jnp.float32)
    ones16 = jnp.ones((16,), jnp.float32)

    for tab in (cnt0, cnt1, fsum, dsum0, dsum1):
      for r in range(34):
        tab[pl.ds(r * 16, 16)] = zeros16
    for k in range(ROW_W // 16):
      pub[pl.ds(k * 16, 16)] = zeros16
      rowbuf[pl.ds(k * 16, 16)] = zeros16

    def start0(k):
      roff = m * 128 + k * 16
      bi = k % 3
      return (pltpu.async_copy(f0.at[b, 0, pl.ds(roff, 16), :],
                               fbufs[bi], semsf[bi]),
              pltpu.async_copy(g0.at[b, 0, pl.ds(roff, 16), :],
                               gbufs[bi], semsg[bi]))

    def stream(body0, body1, d0, stream1):
      if stream1:
        d1f = pltpu.async_copy(f1.at[b, 0, pl.ds(m * 64, 64), :], fb1, semf1)
        d1g = pltpu.async_copy(g1.at[b, 0, pl.ds(m * 64, 64), :], gb1, semg1)
      pend = [d0, start0(1)]
      for k in range(8):
        if k < 6:
          pend.append(start0(k + 2))
        d = pend.pop(0)
        d[0].wait()
        d[1].wait()
        body0(fbufs[k % 3], gbufs[k % 3])
      if stream1:
        d1f.wait()
        d1g.wait()
      body1(fb1, gb1)

    def body_a(cnt, shift, nv=NV):
      mask = (1 << shift) - 1
      def run(fbuf, gbuf):
        @plsc.parallel_loop(0, nv, unroll=8)
        def _(i):
          ri = i >> shift
          ci = (i & mask) * 16
          gv = gbuf[ri, pl.ds(ci, 16)]
          fv = fbuf[ri, pl.ds(ci, 16)]
          idx = lane33 + gv
          plsc.addupdate_scatter(cnt, [idx], ones16)
          plsc.addupdate_scatter(fsum, [idx], fv)
      return run

    stream(body_a(cnt0, 5, 512), body_a(cnt1, 4), start0(0), True)

    def reduce_tab(tab, h):
      acc = tab[pl.ds(h * 16, 16)]
      for r in range(1, 16):
        acc = acc + tab[pl.ds(r * 33 + h * 16, 16)]
      return acc

    db0 = start0(0)
    for h in range(2):
      pub[pl.ds(0 + h * 16, 16)] = reduce_tab(cnt0, h)
      pub[pl.ds(32 + h * 16, 16)] = reduce_tab(cnt1, h)
      pub[pl.ds(64 + h * 16, 16)] = reduce_tab(fsum, h)
    pltpu.sync_copy(pub, shared.at[s])
    plsc.subcore_barrier()

    pltpu.sync_copy(shared.at[pl.ds(lb * GROUP, GROUP)], grp)
    for h in range(2):
      c0c = grp[0, pl.ds(0 + h * 16, 16)]
      c1c = grp[0, pl.ds(32 + h * 16, 16)]
      sfc = grp[0, pl.ds(64 + h * 16, 16)]
      for mm in range(1, GROUP):
        c0c = c0c + grp[mm, pl.ds(0 + h * 16, 16)]
        c1c = c1c + grp[mm, pl.ds(32 + h * 16, 16)]
        sfc = sfc + grp[mm, pl.ds(64 + h * 16, 16)]
      tot = c0c + c1c
      mh = sfc / jnp.maximum(tot, 1.0)
      for l in range(16):
        meantab[pl.ds(l * 33 + h * 16, 16)] = mh
      rowbuf[pl.ds(0 + h * 16, 16)] = c0c
      rowbuf[pl.ds(32 + h * 16, 16)] = c1c
      rowbuf[pl.ds(64 + h * 16, 16)] = sfc

    def body_b(dtab, shift, nv=NV):
      mask = (1 << shift) - 1
      def run(fbuf, gbuf):
        @plsc.parallel_loop(0, nv, unroll=8)
        def _(i):
          ri = i >> shift
          ci = (i & mask) * 16
          gv = gbuf[ri, pl.ds(ci, 16)]
          fv = fbuf[ri, pl.ds(ci, 16)]
          idx = lane33 + gv
          mv = plsc.load_gather(meantab, [idx])
          d = jnp.maximum(jnp.abs(fv - mv) - MARGIN_VAR, 0.0)
          plsc.addupdate_scatter(dtab, [idx], d * d)
      return run

    stream(body_b(dsum0, 5, 512), body_b(dsum1, 4), db0, False)

    for h in range(2):
      pub[pl.ds(96 + h * 16, 16)] = reduce_tab(dsum0, h)
      pub[pl.ds(128 + h * 16, 16)] = reduce_tab(dsum1, h)
    pltpu.sync_copy(pub, shared.at[s])
    plsc.subcore_barrier()

    @pl.when(m == 0)
    def _():
      pltpu.sync_copy(shared.at[pl.ds(lb * GROUP, GROUP)], grp)
      for h in range(2):
        d0c = grp[0, pl.ds(96 + h * 16, 16)]
        d1c = grp[0, pl.ds(128 + h * 16, 16)]
        for mm in range(1, GROUP):
          d0c = d0c + grp[mm, pl.ds(96 + h * 16, 16)]
          d1c = d1c + grp[mm, pl.ds(128 + h * 16, 16)]
        rowbuf[pl.ds(96 + h * 16, 16)] = d0c
        rowbuf[pl.ds(128 + h * 16, 16)] = d1c
      pltpu.sync_copy(rowbuf, out.at[b])

  return sc_kernel


def _epilogue_body(x_ref, o_ref):
  x = x_ref[...]
  c0 = x[:, 0:32]
  c1 = x[:, 32:64]
  sf = x[:, 64:96]
  d0 = x[:, 96:128]
  d1 = x[:, 128:160]
  lanei = lax.broadcasted_iota(jnp.int32, (BATCH, 32), 1)
  tot = c0 + c1
  mean = sf / jnp.maximum(tot, 1.0)
  pres0 = jnp.sum(c0, axis=0, keepdims=True) > 0.0
  big_c = jnp.max(jnp.where(pres0, lanei[0:1, :], 0))
  valid = (tot > 0.0) & (lanei >= 1) & (lanei <= MAX_LABEL) & (lanei <= big_c)
  validf = jnp.where(valid, 1.0, 0.0)
  pull_val = (jnp.where(c0 > 0.0, d0 / jnp.maximum(c0, 1.0), 0.0)
              + jnp.where(c1 > 0.0, d1 / jnp.maximum(c1, 1.0), 0.0))
  pull_sum = jnp.sum(pull_val * validf)
  pull_cnt = jnp.sum(validf)
  push_sum = jnp.float32(0.0)
  push_cnt = jnp.float32(0.0)
  for i in range(1, MAX_LABEL + 1):
    mi = mean[:, i:i + 1]
    vi = validf[:, i:i + 1]
    pv = vi * validf * jnp.where(lanei != i, 1.0, 0.0)
    il = jnp.maximum(2.0 * MARGIN_DIST - jnp.abs(mean - mi), 0.0)
    push_sum = push_sum + jnp.sum(pv * il * il)
    push_cnt = push_cnt + jnp.sum(pv)
  pull = jnp.where(pull_cnt > 0.0,
                   pull_sum / jnp.maximum(pull_cnt, 1.0) * VAR_WEIGHT, 0.0)
  push = jnp.where(push_cnt > 0.0,
                   push_sum / jnp.maximum(push_cnt, 1.0) * DIST_WEIGHT, 0.0)
  o_ref[...] = jnp.full((1, 1), pull + push, jnp.float32)


@jax.jit
def _run(f0, g0, f1, g1):
  rows = _build_sc_kernel()(f0, g0, f1, g1)
  loss = pl.pallas_call(
      _epilogue_body,
      out_shape=jax.ShapeDtypeStruct((1, 1), jnp.float32),
  )(rows)
  return jnp.reshape(loss, ())


def kernel(featmap_s0, featmap_s1, gt_s0, gt_s1):
  g0 = gt_s0.astype(jnp.int32)
  g1 = gt_s1.astype(jnp.int32)
  return _run(featmap_s0, g0, featmap_s1, g1)


# speedup vs baseline: 1.0462x; 1.0114x over previous
import functools

import jax
import jax.numpy as jnp
from jax import lax
from jax.experimental import pallas as pl
from jax.experimental.pallas import tpu as pltpu
from jax.experimental.pallas import tpu_sc as plsc

VAR_WEIGHT = 1.0
DIST_WEIGHT = 1.0
MARGIN_VAR = 0.1
MARGIN_DIST = 1.5
MAX_LABEL = 16

BATCH = 8
GROUP = 4
NV = 1024

ROW_W = 256


@functools.cache
def _build_sc_kernel():
  mesh = plsc.VectorSubcoreMesh(
      core_axis_name="c", subcore_axis_name="s", num_cores=2, num_subcores=16)

  @functools.partial(
      pl.kernel,
      out_type=jax.ShapeDtypeStruct((BATCH, ROW_W), jnp.float32),
      mesh=mesh,
      compiler_params=pltpu.CompilerParams(needs_layout_passes=False),
      scratch_types=[
          pltpu.VMEM((16, 512), jnp.float32),
          pltpu.VMEM((16, 512), jnp.float32),
          pltpu.VMEM((16, 512), jnp.float32),
          pltpu.VMEM((128, 512), jnp.int32),
          pltpu.VMEM((64, 256), jnp.float32),
          pltpu.VMEM((64, 256), jnp.int32),
          pltpu.VMEM((544,), jnp.float32),
          pltpu.VMEM((544,), jnp.float32),
          pltpu.VMEM((544,), jnp.float32),
          pltpu.VMEM((544,), jnp.float32),
          pltpu.VMEM((544,), jnp.float32),
          pltpu.VMEM((544,), jnp.float32),
          pltpu.VMEM((ROW_W,), jnp.float32),
          pltpu.VMEM((ROW_W,), jnp.float32),
          pltpu.VMEM((GROUP, ROW_W), jnp.float32),
          pltpu.VMEM_SHARED((16, ROW_W), jnp.float32),
          pltpu.SemaphoreType.DMA,
          pltpu.SemaphoreType.DMA,
          pltpu.SemaphoreType.DMA,
          pltpu.SemaphoreType.DMA,
          pltpu.SemaphoreType.DMA,
          pltpu.SemaphoreType.DMA,
          pltpu.SemaphoreType.DMA,
          pltpu.SemaphoreType.DMA,
      ],
  )
  def sc_kernel(f0, g0, f1, g1, out, fb0a, fb0b, fb0c, gbig0,
                fb1, gb1, cnt0, cnt1, fsum, dsum0, dsum1, meantab, pub,
                rowbuf, grp, shared, semf0a, semg0a, semf0b, semg0b,
                semf0c, semg0c, semf1, semg1):
    fbufs = (fb0a, fb0b, fb0c)
    semsf = (semf0a, semf0b, semf0c)
    semsg = (semg0a, semg0b, semg0c)
    c = lax.axis_index("c")
    s = lax.axis_index("s")
    lb = s // GROUP
    m = s % GROUP
    b = c * 4 + lb
    lane = lax.iota(jnp.int32, 16)
    lane33 = lane * 33
    zeros16 = jnp.zeros((16,), jnp.float32)
    ones16 = jnp.ones((16,), jnp.float32)

    for tab in (cnt0, cnt1, fsum, dsum0, dsum1):
      for r in range(34):
        tab[pl.ds(r * 16, 16)] = zeros16
    for k in range(ROW_W // 16):
      pub[pl.ds(k * 16, 16)] = zeros16
      rowbuf[pl.ds(k * 16, 16)] = zeros16

    def start0(k, with_g):
      roff = m * 128 + k * 16
      bi = k % 3
      d = [pltpu.async_copy(f0.at[b, 0, pl.ds(roff, 16), :],
                            fbufs[bi], semsf[bi])]
      if with_g:
        d.append(pltpu.async_copy(g0.at[b, 0, pl.ds(roff, 16), :],
                                  gbig0.at[pl.ds(k * 16, 16), :], semsg[bi]))
      return d

    def stream(body0, body1, d0, stream1):
      if stream1:
        d1f = pltpu.async_copy(f1.at[b, 0, pl.ds(m * 64, 64), :], fb1, semf1)
        d1g = pltpu.async_copy(g1.at[b, 0, pl.ds(m * 64, 64), :], gb1, semg1)
      pend = [d0, start0(1, stream1)]
      for k in range(8):
        if k < 6:
          pend.append(start0(k + 2, stream1))
        for d in pend.pop(0):
          d.wait()
        body0(fbufs[k % 3], gbig0, k * 16)
      if stream1:
        d1f.wait()
        d1g.wait()
      body1(fb1, gb1, 0)

    def body_a(cnt, shift, nv=NV):
      mask = (1 << shift) - 1
      def run(fbuf, gref, krow):
        @plsc.parallel_loop(0, nv, unroll=8)
        def _(i):
          ri = i >> shift
          ci = (i & mask) * 16
          gv = gref[krow + ri, pl.ds(ci, 16)]
          fv = fbuf[ri, pl.ds(ci, 16)]
          idx = lane33 + gv
          plsc.addupdate_scatter(cnt, [idx], ones16)
          plsc.addupdate_scatter(fsum, [idx], fv)
      return run

    stream(body_a(cnt0, 5, 512), body_a(cnt1, 4), start0(0, True), True)

    def reduce_tab(tab, h):
      acc = tab[pl.ds(h * 16, 16)]
      for r in range(1, 16):
        acc = acc + tab[pl.ds(r * 33 + h * 16, 16)]
      return acc

    db0 = start0(0, False)
    for h in range(2):
      pub[pl.ds(0 + h * 16, 16)] = reduce_tab(cnt0, h)
      pub[pl.ds(32 + h * 16, 16)] = reduce_tab(cnt1, h)
      pub[pl.ds(64 + h * 16, 16)] = reduce_tab(fsum, h)
    pltpu.sync_copy(pub, shared.at[s])
    plsc.subcore_barrier()

    pltpu.sync_copy(shared.at[pl.ds(lb * GROUP, GROUP)], grp)
    for h in range(2):
      c0c = grp[0, pl.ds(0 + h * 16, 16)]
      c1c = grp[0, pl.ds(32 + h * 16, 16)]
      sfc = grp[0, pl.ds(64 + h * 16, 16)]
      for mm in range(1, GROUP):
        c0c = c0c + grp[mm, pl.ds(0 + h * 16, 16)]
        c1c = c1c + grp[mm, pl.ds(32 + h * 16, 16)]
        sfc = sfc + grp[mm, pl.ds(64 + h * 16, 16)]
      tot = c0c + c1c
      mh = sfc / jnp.maximum(tot, 1.0)
      for l in range(16):
        meantab[pl.ds(l * 33 + h * 16, 16)] = mh
      rowbuf[pl.ds(0 + h * 16, 16)] = c0c
      rowbuf[pl.ds(32 + h * 16, 16)] = c1c
      rowbuf[pl.ds(64 + h * 16, 16)] = sfc

    def body_b(dtab, shift, nv=NV):
      mask = (1 << shift) - 1
      def run(fbuf, gref, krow):
        @plsc.parallel_loop(0, nv, unroll=8)
        def _(i):
          ri = i >> shift
          ci = (i & mask) * 16
          gv = gref[krow + ri, pl.ds(ci, 16)]
          fv = fbuf[ri, pl.ds(ci, 16)]
          idx = lane33 + gv
          mv = plsc.load_gather(meantab, [idx])
          d = jnp.maximum(jnp.abs(fv - mv) - MARGIN_VAR, 0.0)
          plsc.addupdate_scatter(dtab, [idx], d * d)
      return run

    stream(body_b(dsum0, 5, 512), body_b(dsum1, 4), db0, False)

    for h in range(2):
      pub[pl.ds(96 + h * 16, 16)] = reduce_tab(dsum0, h)
      pub[pl.ds(128 + h * 16, 16)] = reduce_tab(dsum1, h)
    pltpu.sync_copy(pub, shared.at[s])
    plsc.subcore_barrier()

    @pl.when(m == 0)
    def _():
      pltpu.sync_copy(shared.at[pl.ds(lb * GROUP, GROUP)], grp)
      for h in range(2):
        d0c = grp[0, pl.ds(96 + h * 16, 16)]
        d1c = grp[0, pl.ds(128 + h * 16, 16)]
        for mm in range(1, GROUP):
          d0c = d0c + grp[mm, pl.ds(96 + h * 16, 16)]
          d1c = d1c + grp[mm, pl.ds(128 + h * 16, 16)]
        rowbuf[pl.ds(96 + h * 16, 16)] = d0c
        rowbuf[pl.ds(128 + h * 16, 16)] = d1c
      pltpu.sync_copy(rowbuf, out.at[b])

  return sc_kernel


def _epilogue_body(x_ref, o_ref):
  x = x_ref[...]
  c0 = x[:, 0:32]
  c1 = x[:, 32:64]
  sf = x[:, 64:96]
  d0 = x[:, 96:128]
  d1 = x[:, 128:160]
  lanei = lax.broadcasted_iota(jnp.int32, (BATCH, 32), 1)
  tot = c0 + c1
  mean = sf / jnp.maximum(tot, 1.0)
  pres0 = jnp.sum(c0, axis=0, keepdims=True) > 0.0
  big_c = jnp.max(jnp.where(pres0, lanei[0:1, :], 0))
  valid = (tot > 0.0) & (lanei >= 1) & (lanei <= MAX_LABEL) & (lanei <= big_c)
  validf = jnp.where(valid, 1.0, 0.0)
  pull_val = (jnp.where(c0 > 0.0, d0 / jnp.maximum(c0, 1.0), 0.0)
              + jnp.where(c1 > 0.0, d1 / jnp.maximum(c1, 1.0), 0.0))
  pull_sum = jnp.sum(pull_val * validf)
  pull_cnt = jnp.sum(validf)
  push_sum = jnp.float32(0.0)
  push_cnt = jnp.float32(0.0)
  for i in range(1, MAX_LABEL + 1):
    mi = mean[:, i:i + 1]
    vi = validf[:, i:i + 1]
    pv = vi * validf * jnp.where(lanei != i, 1.0, 0.0)
    il = jnp.maximum(2.0 * MARGIN_DIST - jnp.abs(mean - mi), 0.0)
    push_sum = push_sum + jnp.sum(pv * il * il)
    push_cnt = push_cnt + jnp.sum(pv)
  pull = jnp.where(pull_cnt > 0.0,
                   pull_sum / jnp.maximum(pull_cnt, 1.0) * VAR_WEIGHT, 0.0)
  push = jnp.where(push_cnt > 0.0,
                   push_sum / jnp.maximum(push_cnt, 1.0) * DIST_WEIGHT, 0.0)
  o_ref[...] = jnp.full((1, 1), pull + push, jnp.float32)


@jax.jit
def _run(f0, g0, f1, g1):
  rows = _build_sc_kernel()(f0, g0, f1, g1)
  loss = pl.pallas_call(
      _epilogue_body,
      out_shape=jax.ShapeDtypeStruct((1, 1), jnp.float32),
  )(rows)
  return jnp.reshape(loss, ())


def kernel(featmap_s0, featmap_s1, gt_s0, gt_s1):
  g0 = gt_s0.astype(jnp.int32)
  g1 = gt_s1.astype(jnp.int32)
  return _run(featmap_s0, g0, featmap_s1, g1)
